# baseline (speedup 1.0000x reference)
import jax
import jax.numpy as jnp
import numpy as np
from jax.experimental import pallas as pl
from jax.experimental.pallas import tpu as pltpu

_N = 5000
_BB = 1280
_NBLK = 4
_NPAD = _BB * _NBLK
_NROUND = _NBLK - 1
_NPAIR = _NBLK // 2
_IOU_THRESH = 0.5


def _round_robin_schedule():
    sched = np.zeros((_NROUND, _NPAIR, 2), dtype=np.int32)
    m = _NBLK - 1
    for r in range(_NROUND):
        pairs = [(m, r)]
        for i in range(1, _NPAIR):
            pairs.append(((r + i) % m, (r - i) % m))
        sched[r] = [(max(a, b), min(a, b)) for a, b in pairs]
    return sched


_SCHED = _round_robin_schedule()


def _iou_block(rows, cols):
    rx1 = rows[:, 0:1]
    ry1 = rows[:, 1:2]
    rx2 = rows[:, 2:3]
    ry2 = rows[:, 3:4]
    cx1 = cols[0:1, :]
    cy1 = cols[1:2, :]
    cx2 = cols[2:3, :]
    cy2 = cols[3:4, :]
    r_area = (rx2 - rx1) * (ry2 - ry1)
    c_area = (cx2 - cx1) * (cy2 - cy1)
    dx = jnp.minimum(rx2, cx2) - jnp.maximum(rx1, cx1)
    h = jnp.maximum(jnp.minimum(ry2, cy2) - jnp.maximum(ry1, cy1), 0.0)
    inter = dx * h
    union = (r_area + c_area) - inter
    return inter / union


def _nms_kernel(sched_ref, rows_ref, out_ref, accr_ref, accc_ref, cols_ref):
    accr_ref[...] = jnp.zeros_like(accr_ref)
    accc_ref[...] = jnp.zeros_like(accc_ref)
    for b in range(_NBLK):
        cols_ref[b] = jnp.transpose(rows_ref[b])

    def _fold128(x):
        m = x[:, 0:128]
        for c in range(1, _BB // 128):
            m = jnp.maximum(m, x[:, c * 128:(c + 1) * 128])
        return m

    def offdiag(hi, lo):
        rows = rows_ref[hi]
        cols = cols_ref[lo]
        rs = rows[:, 4:5]
        cs = cols[4:5, :]
        iou = _iou_block(rows, cols)
        beats = cs >= rs
        rowm = _fold128(jnp.where(beats, iou, 0.0))
        colm = jnp.max(jnp.where(beats, 0.0, iou), axis=0, keepdims=True)
        accr_ref[hi] = jnp.maximum(accr_ref[hi], rowm)
        accc_ref[lo] = jnp.maximum(accc_ref[lo], colm)

    def one_round(r, carry):
        for k in range(_NPAIR):
            offdiag(sched_ref[r, k, 0], sched_ref[r, k, 1])
        return 0

    jax.lax.fori_loop(0, _NROUND, one_round, 0)

    tie = (jax.lax.broadcasted_iota(jnp.int32, (1, _BB), 1) <
           jax.lax.broadcasted_iota(jnp.int32, (_BB, 1), 0))
    eye = (jax.lax.broadcasted_iota(jnp.int32, (_BB, _BB), 0) ==
           jax.lax.broadcasted_iota(jnp.int32, (_BB, _BB), 1)).astype(
               jnp.float32)

    def finish(b, carry):
        rows = rows_ref[b]
        cols = cols_ref[b]
        rs = rows[:, 4:5]
        cs = cols[4:5, :]
        iou = _iou_block(rows, cols)
        beats_d = (cs > rs) | ((cs == rs) & tie)
        md = jnp.where(beats_d, iou, 0.0)
        m = jnp.maximum(accr_ref[b], md[:, 0:128])
        for c in range(1, _BB // 128):
            m = jnp.maximum(m, md[:, c * 128:(c + 1) * 128])
        rowm = jnp.max(m, axis=1, keepdims=True)
        cc = jnp.sum(eye * accc_ref[b], axis=1, keepdims=True)
        sup = jnp.maximum(rowm, cc) > _IOU_THRESH
        out_ref[b] = jnp.where(sup, 0.0, rows)
        return 0

    jax.lax.fori_loop(0, _NBLK, finish, 0)


def kernel(boxes, scores):
    feat = jnp.concatenate(
        [boxes, scores[:, None], jnp.zeros((_N, 3), jnp.float32)], axis=1)
    data = jnp.pad(feat, ((0, _NPAD - _N), (0, 0)))
    rows = data.reshape(_NBLK, _BB, 8)
    sched = jnp.asarray(_SCHED)

    out = pl.pallas_call(
        _nms_kernel,
        in_specs=[
            pl.BlockSpec(memory_space=pltpu.MemorySpace.SMEM),
            pl.BlockSpec(memory_space=pltpu.MemorySpace.VMEM),
        ],
        out_specs=pl.BlockSpec(memory_space=pltpu.MemorySpace.VMEM),
        out_shape=jax.ShapeDtypeStruct((_NBLK, _BB, 8), jnp.float32),
        scratch_shapes=[
            pltpu.VMEM((_NBLK, _BB, 128), jnp.float32),
            pltpu.VMEM((_NBLK, 1, _BB), jnp.float32),
            pltpu.VMEM((_NBLK, 8, _BB), jnp.float32),
        ],
    )(sched, rows)

    return out.reshape(_NPAD, 8)[:_N, :5]


# speedup vs baseline: 1.0143x; 1.0143x over previous
import jax
import jax.numpy as jnp
import numpy as np
from jax.experimental import pallas as pl
from jax.experimental.pallas import tpu as pltpu

_N = 5000
_BB = 1024
_NBLK = 5
_NPAD = _BB * _NBLK
_NROUND = _NBLK
_NPAIR = (_NBLK - 1) // 2
_IOU_THRESH = 0.5


def _round_robin_schedule():
    sched = np.zeros((_NROUND, _NPAIR, 2), dtype=np.int32)
    for r in range(_NROUND):
        pairs = []
        for i in range(1, _NPAIR + 1):
            pairs.append(((r + i) % _NBLK, (r - i) % _NBLK))
        sched[r] = [(max(a, b), min(a, b)) for a, b in pairs]
    return sched


_SCHED = _round_robin_schedule()


def _iou_block(rows, cols):
    rx1 = rows[:, 0:1]
    ry1 = rows[:, 1:2]
    rx2 = rows[:, 2:3]
    ry2 = rows[:, 3:4]
    cx1 = cols[0:1, :]
    cy1 = cols[1:2, :]
    cx2 = cols[2:3, :]
    cy2 = cols[3:4, :]
    r_area = (rx2 - rx1) * (ry2 - ry1)
    c_area = (cx2 - cx1) * (cy2 - cy1)
    dx = jnp.minimum(rx2, cx2) - jnp.maximum(rx1, cx1)
    h = jnp.maximum(jnp.minimum(ry2, cy2) - jnp.maximum(ry1, cy1), 0.0)
    inter = dx * h
    union = (r_area + c_area) - inter
    return inter / union


def _nms_kernel(sched_ref, rows_ref, cols_ref, out_ref, accr_ref, accc_ref):
    accr_ref[...] = jnp.zeros_like(accr_ref)
    accc_ref[...] = jnp.zeros_like(accc_ref)

    def _fold128(x):
        m = x[:, 0:128]
        for c in range(1, _BB // 128):
            m = jnp.maximum(m, x[:, c * 128:(c + 1) * 128])
        return m

    def offdiag(hi, lo):
        rows = rows_ref[hi]
        cols = cols_ref[lo]
        rs = rows[:, 4:5]
        cs = cols[4:5, :]
        iou = _iou_block(rows, cols)
        beats = cs >= rs
        rowm = _fold128(jnp.where(beats, iou, 0.0))
        colm = jnp.max(jnp.where(beats, 0.0, iou), axis=0, keepdims=True)
        accr_ref[hi] = jnp.maximum(accr_ref[hi], rowm)
        accc_ref[lo] = jnp.maximum(accc_ref[lo], colm)

    def one_round(r, carry):
        for k in range(_NPAIR):
            offdiag(sched_ref[r, k, 0], sched_ref[r, k, 1])
        return 0

    jax.lax.fori_loop(0, _NROUND, one_round, 0)

    tie = (jax.lax.broadcasted_iota(jnp.int32, (1, _BB), 1) <
           jax.lax.broadcasted_iota(jnp.int32, (_BB, 1), 0))
    eye = (jax.lax.broadcasted_iota(jnp.int32, (_BB, _BB), 0) ==
           jax.lax.broadcasted_iota(jnp.int32, (_BB, _BB), 1)).astype(
               jnp.float32)

    def finish(b, carry):
        rows = rows_ref[b]
        cols = cols_ref[b]
        rs = rows[:, 4:5]
        cs = cols[4:5, :]
        iou = _iou_block(rows, cols)
        beats_d = (cs > rs) | ((cs == rs) & tie)
        md = jnp.where(beats_d, iou, 0.0)
        m = jnp.maximum(accr_ref[b], md[:, 0:128])
        for c in range(1, _BB // 128):
            m = jnp.maximum(m, md[:, c * 128:(c + 1) * 128])
        rowm = jnp.max(m, axis=1, keepdims=True)
        cc = jnp.sum(eye * accc_ref[b], axis=1, keepdims=True)
        sup = jnp.maximum(rowm, cc) > _IOU_THRESH
        out_ref[b] = jnp.where(sup, 0.0, rows)
        return 0

    jax.lax.fori_loop(0, _NBLK, finish, 0)


def kernel(boxes, scores):
    feat = jnp.concatenate([boxes, scores[:, None]], axis=1)
    data = jnp.pad(feat, ((0, _NPAD - _N), (0, 3)))
    rows = data.reshape(_NBLK, _BB, 8)
    cols = jnp.swapaxes(rows, 1, 2)
    sched = jnp.asarray(_SCHED)

    out = pl.pallas_call(
        _nms_kernel,
        in_specs=[
            pl.BlockSpec(memory_space=pltpu.MemorySpace.SMEM),
            pl.BlockSpec(memory_space=pltpu.MemorySpace.VMEM),
            pl.BlockSpec(memory_space=pltpu.MemorySpace.VMEM),
        ],
        out_specs=pl.BlockSpec(memory_space=pltpu.MemorySpace.VMEM),
        out_shape=jax.ShapeDtypeStruct((_NBLK, _BB, 8), jnp.float32),
        scratch_shapes=[
            pltpu.VMEM((_NBLK, _BB, 128), jnp.float32),
            pltpu.VMEM((_NBLK, 1, _BB), jnp.float32),
        ],
    )(sched, rows, cols)

    return out.reshape(_NPAD, 8)[:_N, :5]


# speedup vs baseline: 1.0390x; 1.0244x over previous
import jax
import jax.numpy as jnp
import numpy as np
from jax.experimental import pallas as pl
from jax.experimental.pallas import tpu as pltpu

_N = 5000
_BB = 1024
_NBLK = 5
_NPAD = _BB * _NBLK
_NROUND = _NBLK
_NPAIR = (_NBLK - 1) // 2
_IOU_THRESH = 0.5


def _round_robin_schedule():
    sched = np.zeros((_NROUND, _NPAIR, 2), dtype=np.int32)
    for r in range(_NROUND):
        pairs = []
        for i in range(1, _NPAIR + 1):
            pairs.append(((r + i) % _NBLK, (r - i) % _NBLK))
        sched[r] = [(max(a, b), min(a, b)) for a, b in pairs]
    return sched


_SCHED = _round_robin_schedule()


def _iou_block(rows, cols):
    rx1 = rows[:, 0:1]
    ry1 = rows[:, 1:2]
    rx2 = rows[:, 2:3]
    ry2 = rows[:, 3:4]
    cx1 = cols[0:1, :]
    cy1 = cols[1:2, :]
    cx2 = cols[2:3, :]
    cy2 = cols[3:4, :]
    r_area = (rx2 - rx1) * (ry2 - ry1)
    c_area = (cx2 - cx1) * (cy2 - cy1)
    dx = jnp.minimum(rx2, cx2) - jnp.maximum(rx1, cx1)
    h = jnp.maximum(jnp.minimum(ry2, cy2) - jnp.maximum(ry1, cy1), 0.0)
    inter = dx * h
    union = (r_area + c_area) - inter
    return inter / union


def _nms_kernel(sched_ref, rows_ref, cols_ref, out_ref, accr_ref, accc_ref):
    accr_ref[...] = jnp.zeros_like(accr_ref)
    accc_ref[...] = jnp.zeros_like(accc_ref)

    def _fold128(x):
        m = x[:, 0:128]
        for c in range(1, _BB // 128):
            m = jnp.maximum(m, x[:, c * 128:(c + 1) * 128])
        return m

    def offdiag(hi, lo):
        rows = rows_ref[hi]
        cols = cols_ref[lo]
        rs = rows[:, 4:5]
        cs = cols[4:5, :]
        iou = _iou_block(rows, cols)
        beats = cs >= rs
        rowm = _fold128(jnp.where(beats, iou, 0.0))
        colm = jnp.max(jnp.where(beats, 0.0, iou), axis=0, keepdims=True)
        accr_ref[hi] = jnp.maximum(accr_ref[hi], rowm)
        accc_ref[lo] = jnp.maximum(accc_ref[lo], colm)

    def one_round(r, carry):
        for k in range(_NPAIR):
            offdiag(sched_ref[r, k, 0], sched_ref[r, k, 1])
        return 0

    jax.lax.fori_loop(0, _NROUND, one_round, 0)

    tie = (jax.lax.broadcasted_iota(jnp.int32, (1, _BB), 1) <
           jax.lax.broadcasted_iota(jnp.int32, (_BB, 1), 0))
    def finish(b, carry):
        rows = rows_ref[b]
        cols = cols_ref[b]
        rs = rows[:, 4:5]
        cs = cols[4:5, :]
        iou = _iou_block(rows, cols)
        beats_d = (cs > rs) | ((cs == rs) & tie)
        md = jnp.where(beats_d, iou, 0.0)
        m = jnp.maximum(accr_ref[b], md[:, 0:128])
        for c in range(1, _BB // 128):
            m = jnp.maximum(m, md[:, c * 128:(c + 1) * 128])
        rowm = jnp.max(m, axis=1, keepdims=True)
        cc = jnp.transpose(accc_ref[b])
        sup = jnp.maximum(rowm, cc) > _IOU_THRESH
        out_ref[b] = jnp.where(sup, 0.0, rows)
        return 0

    jax.lax.fori_loop(0, _NBLK, finish, 0)


def kernel(boxes, scores):
    feat = jnp.concatenate([boxes, scores[:, None]], axis=1)
    data = jnp.pad(feat, ((0, _NPAD - _N), (0, 3)))
    rows = data.reshape(_NBLK, _BB, 8)
    cols = jnp.swapaxes(rows, 1, 2)
    sched = jnp.asarray(_SCHED)

    out = pl.pallas_call(
        _nms_kernel,
        in_specs=[
            pl.BlockSpec(memory_space=pltpu.MemorySpace.SMEM),
            pl.BlockSpec(memory_space=pltpu.MemorySpace.VMEM),
            pl.BlockSpec(memory_space=pltpu.MemorySpace.VMEM),
        ],
        out_specs=pl.BlockSpec(memory_space=pltpu.MemorySpace.VMEM),
        out_shape=jax.ShapeDtypeStruct((_NBLK, _BB, 8), jnp.float32),
        scratch_shapes=[
            pltpu.VMEM((_NBLK, _BB, 128), jnp.float32),
            pltpu.VMEM((_NBLK, 1, _BB), jnp.float32),
        ],
    )(sched, rows, cols)

    return out.reshape(_NPAD, 8)[:_N, :5]
